# baseline (device time: 13546 ns/iter reference)
import jax
import jax.numpy as jnp
from jax import lax
from jax.experimental import pallas as pl
from jax.experimental.pallas import tpu as pltpu

N_DEV = 4


def kernel(x, pi):
    _, m, n = x.shape

    def body(
        pi_ref,
        x_ref,
        out_ref,
        xv_ref,
        comm_ref,
        copy_sem,
        send_sem,
        recv_sem,
        ack_sem,
    ):
        my_i = lax.axis_index("i")
        dst = pi_ref[my_i]
        src = jnp.int32(0)
        for j in range(N_DEV):
            src += jnp.int32(j) * (pi_ref[j] == my_i).astype(jnp.int32)

        barrier_sem = pltpu.get_barrier_semaphore()
        for peer in (dst, src):
            pl.semaphore_signal(
                barrier_sem,
                inc=1,
                device_id=(peer,),
                device_id_type=pl.DeviceIdType.MESH,
            )

        cp = pltpu.make_async_copy(x_ref.at[0], xv_ref, copy_sem)
        cp.start()
        cp.wait()
        comm_ref[...] = xv_ref[...].astype(jnp.bfloat16)

        pl.semaphore_wait(barrier_sem, 2)

        rdma = pltpu.make_async_remote_copy(
            src_ref=comm_ref,
            dst_ref=out_ref.at[0],
            send_sem=send_sem,
            recv_sem=recv_sem,
            device_id=(dst,),
            device_id_type=pl.DeviceIdType.MESH,
        )
        rdma.start()
        rdma.wait()

        @pl.when(dst != src)
        def _():
            pl.semaphore_signal(
                ack_sem,
                inc=1,
                device_id=(src,),
                device_id_type=pl.DeviceIdType.MESH,
            )
            pl.semaphore_wait(ack_sem, 1)

    grid_spec = pltpu.PrefetchScalarGridSpec(
        num_scalar_prefetch=1,
        grid=(),
        in_specs=[
            pl.BlockSpec(memory_space=pl.ANY),
        ],
        out_specs=pl.BlockSpec(memory_space=pl.ANY),
        scratch_shapes=[
            pltpu.VMEM((m, n), jnp.float32),
            pltpu.VMEM((m, n), jnp.bfloat16),
            pltpu.SemaphoreType.DMA,
            pltpu.SemaphoreType.DMA,
            pltpu.SemaphoreType.DMA,
            pltpu.SemaphoreType.REGULAR,
        ],
    )
    return pl.pallas_call(
        body,
        grid_spec=grid_spec,
        out_shape=jax.ShapeDtypeStruct((1, m, n), jnp.bfloat16),
        compiler_params=pltpu.CompilerParams(collective_id=0),
    )(pi, x)


# device time: 13206 ns/iter; 1.0257x vs baseline; 1.0257x over previous
import jax
import jax.numpy as jnp
from jax import lax
from jax.experimental import pallas as pl
from jax.experimental.pallas import tpu as pltpu

N_DEV = 4


def kernel(x, pi):
    _, m, n = x.shape

    def body(pi_ref, x_ref, out_ref, comm_ref, send_sem, recv_sem, ack_sem):
        my_i = lax.axis_index("i")
        dst = pi_ref[my_i]
        src = jnp.int32(0)
        for j in range(N_DEV):
            src += jnp.int32(j) * (pi_ref[j] == my_i).astype(jnp.int32)

        barrier_sem = pltpu.get_barrier_semaphore()
        for peer in (dst, src):
            pl.semaphore_signal(
                barrier_sem,
                inc=1,
                device_id=(peer,),
                device_id_type=pl.DeviceIdType.MESH,
            )

        comm_ref[...] = x_ref[0].astype(jnp.bfloat16)

        pl.semaphore_wait(barrier_sem, 2)

        rdma = pltpu.make_async_remote_copy(
            src_ref=comm_ref,
            dst_ref=out_ref.at[0],
            send_sem=send_sem,
            recv_sem=recv_sem,
            device_id=(dst,),
            device_id_type=pl.DeviceIdType.MESH,
        )
        rdma.start()
        rdma.wait()

        @pl.when(dst != src)
        def _():
            pl.semaphore_signal(
                ack_sem,
                inc=1,
                device_id=(src,),
                device_id_type=pl.DeviceIdType.MESH,
            )
            pl.semaphore_wait(ack_sem, 1)

    return pl.pallas_call(
        body,
        out_shape=jax.ShapeDtypeStruct((1, m, n), jnp.bfloat16),
        in_specs=[
            pl.BlockSpec(memory_space=pltpu.SMEM),
            pl.BlockSpec(memory_space=pltpu.VMEM),
        ],
        out_specs=pl.BlockSpec(memory_space=pltpu.VMEM),
        scratch_shapes=[
            pltpu.VMEM((m, n), jnp.bfloat16),
            pltpu.SemaphoreType.DMA,
            pltpu.SemaphoreType.DMA,
            pltpu.SemaphoreType.REGULAR,
        ],
        compiler_params=pltpu.CompilerParams(collective_id=0),
    )(pi, x)
